# baseline (speedup 1.0000x reference)
import functools

import jax
import jax.numpy as jnp
from jax import lax
from jax.experimental import pallas as pl
from jax.experimental.pallas import tpu as pltpu
from jax.experimental.pallas import tpu_sc as plsc

EMBED = 768
HIDDEN = 3072
E = 8
T = 2048
M = 128
NB = (2 * T) // M + E
P_MAX = NB * M
NC, NS = 2, 16
NW = NC * NS
RPW = P_MAX // NW
GCHUNK = 80
TPW = T // NW
CCHUNK = 32
CH = 512


def _router_body(x_ref, wg_ref, pos0_ref, pos1_ref, wgt0_ref, wgt1_ref,
                 be_ref, va_ref):
    x = x_ref[...]
    logits = lax.dot_general(x, wg_ref[...], (((1,), (1,)), ((), ())),
                             preferred_element_type=jnp.float32)
    mx = jnp.max(logits, axis=1, keepdims=True)
    ex = jnp.exp(logits - mx)
    probs = ex / jnp.sum(ex, axis=1, keepdims=True)

    r8 = lax.broadcasted_iota(jnp.int32, (E, E), 0)
    c8 = lax.broadcasted_iota(jnp.int32, (E, E), 1)
    u_incl = (r8 <= c8).astype(jnp.float32)
    u_excl = (r8 < c8).astype(jnp.float32)

    p1 = jnp.max(probs, axis=1, keepdims=True)
    eq1 = (probs == p1).astype(jnp.float32)
    cs1 = lax.dot_general(eq1, u_incl, (((1,), (0,)), ((), ())),
                          preferred_element_type=jnp.float32)
    m1 = (eq1 > 0) & (cs1 == 1.0)
    probs2 = jnp.where(m1, -jnp.inf, probs)
    p2 = jnp.max(probs2, axis=1, keepdims=True)
    eq2 = (probs2 == p2).astype(jnp.float32)
    cs2 = lax.dot_general(eq2, u_incl, (((1,), (0,)), ((), ())),
                          preferred_element_type=jnp.float32)
    m2 = (eq2 > 0) & (cs2 == 1.0)

    denom = p1 + p2 + 1e-9
    wgt0_ref[...] = p1 / denom
    wgt1_ref[...] = p2 / denom

    oh0 = m1.astype(jnp.float32)
    oh1 = m2.astype(jnp.float32)
    oh = jnp.concatenate([oh0, oh1], axis=1)

    rl = lax.broadcasted_iota(jnp.int32, (CH, CH), 0)
    cl = lax.broadcasted_iota(jnp.int32, (CH, CH), 1)
    l_strict = (rl > cl).astype(jnp.float32)
    carry = jnp.zeros((1, 2 * E), jnp.float32)
    cum_chunks = []
    for i in range(T // CH):
        chunk = oh[i * CH:(i + 1) * CH, :]
        part = lax.dot_general(l_strict, chunk, (((1,), (0,)), ((), ())),
                               preferred_element_type=jnp.float32)
        cum_chunks.append(part + carry)
        carry = carry + jnp.sum(chunk, axis=0, keepdims=True)
    cum = jnp.concatenate(cum_chunks, axis=0)

    cnt0 = carry[:, :E]
    counts = carry[:, :E] + carry[:, E:]
    fm = float(M)
    nb = jnp.floor((counts + (fm - 1.0)) / fm)
    pc = nb * fm
    off = lax.dot_general(pc, u_excl, (((1,), (0,)), ((), ())),
                          preferred_element_type=jnp.float32)
    boff = lax.dot_general(nb, u_excl, (((1,), (0,)), ((), ())),
                           preferred_element_type=jnp.float32)
    total_nb = jnp.sum(nb, axis=1, keepdims=True)

    rank0 = jnp.sum(oh0 * cum[:, :E], axis=1, keepdims=True)
    rank1 = jnp.sum(oh1 * (cum[:, E:] + cnt0), axis=1, keepdims=True)
    pos0_ref[...] = (jnp.sum(oh0 * off, axis=1, keepdims=True)
                     + rank0).astype(jnp.int32)
    pos1_ref[...] = (jnp.sum(oh1 * off, axis=1, keepdims=True)
                     + rank1).astype(jnp.int32)

    bi = lax.broadcasted_iota(jnp.int32, (NB, 1), 0).astype(jnp.float32)
    ge = (bi >= boff).astype(jnp.float32)
    be_ref[...] = (jnp.sum(ge, axis=1, keepdims=True) - 1.0).astype(jnp.int32)
    va_ref[...] = (bi < total_nb).astype(jnp.int32)


def _router(xf, Wg):
    return pl.pallas_call(
        _router_body,
        out_shape=[
            jax.ShapeDtypeStruct((T, 1), jnp.int32),
            jax.ShapeDtypeStruct((T, 1), jnp.int32),
            jax.ShapeDtypeStruct((T, 1), jnp.float32),
            jax.ShapeDtypeStruct((T, 1), jnp.float32),
            jax.ShapeDtypeStruct((NB, 1), jnp.int32),
            jax.ShapeDtypeStruct((NB, 1), jnp.int32),
        ],
    )(xf, Wg)


def _dispatch(pos0, pos1, wgt0, wgt1, xf):
    mesh = plsc.VectorSubcoreMesh(core_axis_name="c", subcore_axis_name="s")

    @functools.partial(
        pl.kernel,
        out_type=[
            jax.ShapeDtypeStruct((P_MAX, EMBED), jnp.float32),
            jax.ShapeDtypeStruct((P_MAX,), jnp.float32),
        ],
        mesh=mesh,
        compiler_params=pltpu.CompilerParams(needs_layout_passes=False),
        scratch_types=[
            pltpu.VMEM((T,), jnp.int32),
            pltpu.VMEM((T,), jnp.float32),
            pltpu.VMEM((P_MAX,), jnp.int32),
            pltpu.VMEM((P_MAX,), jnp.float32),
            pltpu.VMEM_SHARED((P_MAX,), jnp.int32),
            pltpu.VMEM((RPW // GCHUNK, GCHUNK), jnp.int32),
            pltpu.VMEM((GCHUNK, EMBED), jnp.float32),
            pltpu.SemaphoreType.DMA,
        ],
    )
    def dispatch_k(pos0_ref, pos1_ref, wgt0_ref, wgt1_ref, x_ref,
                   xs_ref, wrow_ref,
                   posb, wgtb, tokf, wrf, toks, idxv, rowbuf, sem):
        cid = lax.axis_index("c")
        sid = lax.axis_index("s")
        wid = sid * NC + cid

        @pl.when(sid == 0)
        def _scatter():
            zi = jnp.zeros((16,), jnp.int32)
            zf = jnp.zeros((16,), jnp.float32)

            def zstep(j, _):
                tokf[pl.ds(j * 16, 16)] = zi
                wrf[pl.ds(j * 16, 16)] = zf
                return 0
            lax.fori_loop(0, P_MAX // 16, zstep, 0)

            for pref, wref in ((pos0_ref, wgt0_ref), (pos1_ref, wgt1_ref)):
                pltpu.sync_copy(pref, posb)
                pltpu.sync_copy(wref, wgtb)

                def sstep(j, _):
                    idx = posb[pl.ds(j * 16, 16)]
                    tv = j * 16 + lax.broadcasted_iota(jnp.int32, (16,), 0)
                    plsc.store_scatter(tokf, [idx], tv)
                    wv = wgtb[pl.ds(j * 16, 16)]
                    plsc.store_scatter(wrf, [idx], wv)
                    return 0
                lax.fori_loop(0, T // 16, sstep, 0)

            pltpu.sync_copy(tokf, toks)

            @pl.when(cid == 0)
            def _():
                pltpu.sync_copy(wrf, wrow_ref)

        plsc.subcore_barrier()

        base = wid * RPW
        for g in range(RPW // GCHUNK):
            pltpu.sync_copy(toks.at[pl.ds(base + g * GCHUNK, GCHUNK)],
                            idxv.at[g])
        for g in range(RPW // GCHUNK):
            pltpu.async_copy(x_ref.at[idxv.at[g]], rowbuf, sem).wait()
            pltpu.sync_copy(rowbuf,
                            xs_ref.at[pl.ds(base + g * GCHUNK, GCHUNK)])

    return dispatch_k(pos0, pos1, wgt0, wgt1, xf)


def _ffn_body(be_ref, va_ref, xs_ref, wr_ref, w1_ref, b1_ref, w2_ref, b2_ref,
              ys_ref):
    i = pl.program_id(0)

    @pl.when(va_ref[i] != 0)
    def _():
        h = lax.dot_general(xs_ref[...], w1_ref[0], (((1,), (1,)), ((), ())),
                            preferred_element_type=jnp.float32)
        h = jnp.maximum(h + b1_ref[0], 0.0)
        y = lax.dot_general(h, w2_ref[0], (((1,), (1,)), ((), ())),
                            preferred_element_type=jnp.float32)
        ys_ref[...] = (y + b2_ref[0]) * wr_ref[...]


def _ffn(be, va, xs, wrow2d, W1, b1r, W2, b2r):
    grid_spec = pltpu.PrefetchScalarGridSpec(
        num_scalar_prefetch=2,
        grid=(NB,),
        in_specs=[
            pl.BlockSpec((M, EMBED), lambda i, be, va: (i, 0)),
            pl.BlockSpec((M, 1), lambda i, be, va: (i, 0)),
            pl.BlockSpec((1, HIDDEN, EMBED), lambda i, be, va: (be[i], 0, 0)),
            pl.BlockSpec((1, 1, HIDDEN), lambda i, be, va: (be[i], 0, 0)),
            pl.BlockSpec((1, EMBED, HIDDEN), lambda i, be, va: (be[i], 0, 0)),
            pl.BlockSpec((1, 1, EMBED), lambda i, be, va: (be[i], 0, 0)),
        ],
        out_specs=pl.BlockSpec((M, EMBED), lambda i, be, va: (i, 0)),
    )
    return pl.pallas_call(
        _ffn_body,
        grid_spec=grid_spec,
        out_shape=jax.ShapeDtypeStruct((P_MAX, EMBED), jnp.float32),
    )(be, va, xs, wrow2d, W1, b1r, W2, b2r)


def _combine(pos0, pos1, ys):
    mesh = plsc.VectorSubcoreMesh(core_axis_name="c", subcore_axis_name="s")

    @functools.partial(
        pl.kernel,
        out_type=jax.ShapeDtypeStruct((T, EMBED), jnp.float32),
        mesh=mesh,
        compiler_params=pltpu.CompilerParams(needs_layout_passes=False),
        scratch_types=[
            pltpu.VMEM((TPW // CCHUNK, CCHUNK), jnp.int32),
            pltpu.VMEM((TPW // CCHUNK, CCHUNK), jnp.int32),
            pltpu.VMEM((CCHUNK, EMBED), jnp.float32),
            pltpu.VMEM((CCHUNK, EMBED), jnp.float32),
            pltpu.SemaphoreType.DMA,
            pltpu.SemaphoreType.DMA,
        ],
    )
    def combine_k(pos0_ref, pos1_ref, ys_ref, out_ref,
                  p0v, p1v, buf0, buf1, sem0, sem1):
        cid = lax.axis_index("c")
        sid = lax.axis_index("s")
        wid = sid * NC + cid
        tbase = wid * TPW

        nch = TPW // CCHUNK
        for c in range(nch):
            pltpu.sync_copy(pos0_ref.at[pl.ds(tbase + c * CCHUNK, CCHUNK)],
                            p0v.at[c])
            pltpu.sync_copy(pos1_ref.at[pl.ds(tbase + c * CCHUNK, CCHUNK)],
                            p1v.at[c])

        for c in range(nch):
            a = pltpu.async_copy(ys_ref.at[p0v.at[c]], buf0, sem0)
            b = pltpu.async_copy(ys_ref.at[p1v.at[c]], buf1, sem1)
            a.wait()
            b.wait()

            def row_add(i, _):
                for j in range(EMBED // 16):
                    sl = pl.ds(j * 16, 16)
                    buf0[i, sl] = buf0[i, sl] + buf1[i, sl]
                return 0
            lax.fori_loop(0, CCHUNK, row_add, 0)

            pltpu.sync_copy(buf0,
                            out_ref.at[pl.ds(tbase + c * CCHUNK, CCHUNK)])

    return combine_k(pos0, pos1, ys)


def kernel(x, Wg, W1, b1, W2, b2):
    orig_shape = x.shape
    xf = x.reshape(T, EMBED)
    pos0, pos1, wgt0, wgt1, be, va = _router(xf, Wg)
    pos0 = pos0.reshape(T)
    pos1 = pos1.reshape(T)
    xs, wrow = _dispatch(pos0, pos1, wgt0.reshape(T), wgt1.reshape(T), xf)
    ys = _ffn(be.reshape(NB), va.reshape(NB), xs, wrow.reshape(P_MAX, 1),
              W1, b1.reshape(E, 1, HIDDEN), W2, b2.reshape(E, 1, EMBED))
    out = _combine(pos0, pos1, ys)
    return out.reshape(orig_shape)
